# baseline (speedup 1.0000x reference)
import functools

import jax
import jax.numpy as jnp
from jax import lax
from jax.experimental import pallas as pl
from jax.experimental.pallas import tpu as pltpu
from jax.experimental.pallas import tpu_sc as plsc

VOCAB = 1000000
DIM = 64
N = 16384

NC = 2
NS = 16
NW = NC * NS
B_W = N // NW
CHUNK = 128
NCHUNK = B_W // CHUNK


@functools.partial(
    pl.kernel,
    mesh=plsc.VectorSubcoreMesh(core_axis_name="c", subcore_axis_name="s"),
    out_type=jax.ShapeDtypeStruct((NW, DIM), jnp.float32),
    scratch_types=[
        pltpu.VMEM((NCHUNK, CHUNK), jnp.int32),
        pltpu.VMEM((B_W + 16,), jnp.int32),
        pltpu.VMEM((B_W, 2 * DIM), jnp.float32),
        pltpu.VMEM((DIM,), jnp.float32),
        pltpu.SemaphoreType.DMA,
    ],
)
def _gather_partials(idx_hbm, off_hbm, table_hbm, out_hbm,
                     idx_v, off_v, rows_v, acc_v, sem):
    wid = lax.axis_index("s") * NC + lax.axis_index("c")
    pltpu.sync_copy(idx_hbm.at[wid], idx_v)
    pltpu.sync_copy(off_hbm.at[wid], off_v.at[pl.ds(0, B_W)])
    copies = [
        pltpu.async_copy(
            table_hbm.at[idx_v.at[j]],
            rows_v.at[pl.ds(j * CHUNK, CHUNK)],
            sem,
        )
        for j in range(NCHUNK)
    ]
    for c in copies:
        c.wait()

    def body(i, accs):
        off = off_v[pl.ds(i, 16)][0]
        return tuple(
            accs[k] + rows_v[i, pl.ds(off + k * 16, 16)]
            for k in range(DIM // 16)
        )

    zeros = tuple(jnp.zeros((16,), jnp.float32) for _ in range(DIM // 16))
    accs = lax.fori_loop(0, B_W, body, zeros)
    for k in range(DIM // 16):
        acc_v[pl.ds(k * 16, 16)] = accs[k]
    pltpu.sync_copy(acc_v, out_hbm.at[wid])


def _finalize_body(p_ref, w_ref, o_ref):
    s = jnp.sum(p_ref[...], axis=0, keepdims=True)
    o_ref[...] = jnp.sum(s * w_ref[...], axis=1, keepdims=True) * (1.0 / N)


_finalize = pl.pallas_call(
    _finalize_body,
    out_shape=jax.ShapeDtypeStruct((1, 1), jnp.float32),
)


def kernel(word_idxs, table, weights):
    idx = word_idxs.astype(jnp.int32)
    pair = (idx >> 1).reshape(NW, NCHUNK, CHUNK)
    off = ((idx & 1) << 6).reshape(NW, B_W)
    pairs_view = table.reshape(VOCAB // 2, 2 * DIM)
    partials = _gather_partials(pair, off, pairs_view)
    prod = _finalize(partials, weights.reshape(1, DIM))
    return jnp.reshape(prod, ())


# speedup vs baseline: 1.7219x; 1.7219x over previous
import functools

import jax
import jax.numpy as jnp
from jax import lax
from jax.experimental import pallas as pl
from jax.experimental.pallas import tpu as pltpu
from jax.experimental.pallas import tpu_sc as plsc

VOCAB = 1000000
DIM = 64
N = 16384

NC = 2
NS = 16
NW = NC * NS
B_W = N // NW
UNROLL = 16


@functools.partial(
    pl.kernel,
    mesh=plsc.VectorSubcoreMesh(core_axis_name="c", subcore_axis_name="s"),
    out_type=jax.ShapeDtypeStruct((NW, DIM), jnp.float32),
    scratch_types=[
        pltpu.VMEM((B_W,), jnp.int32),
        pltpu.VMEM((B_W, DIM), jnp.float32),
        pltpu.VMEM((DIM,), jnp.float32),
        pltpu.SemaphoreType.DMA,
    ],
)
def _gather_partials(idx_hbm, table_hbm, out_hbm, idx_v, rows_v, acc_v, sem):
    wid = lax.axis_index("s") * NC + lax.axis_index("c")
    pltpu.sync_copy(idx_hbm.at[wid], idx_v)

    def enqueue(step, _):
        base = step * UNROLL
        vec = idx_v[pl.ds(base, UNROLL)]
        for j in range(UNROLL):
            pltpu.async_copy(table_hbm.at[vec[j]], rows_v.at[base + j], sem)
        return 0

    lax.fori_loop(0, B_W // UNROLL, enqueue, 0)
    pltpu.make_async_copy(table_hbm.at[pl.ds(0, B_W)], rows_v, sem).wait()

    def body(i, accs):
        return tuple(
            accs[k] + rows_v[i, pl.ds(k * 16, 16)] for k in range(DIM // 16)
        )

    zeros = tuple(jnp.zeros((16,), jnp.float32) for _ in range(DIM // 16))
    accs = lax.fori_loop(0, B_W, body, zeros)
    for k in range(DIM // 16):
        acc_v[pl.ds(k * 16, 16)] = accs[k]
    pltpu.sync_copy(acc_v, out_hbm.at[wid])


def _finalize_body(p_ref, w_ref, o_ref):
    s = jnp.sum(p_ref[...], axis=0, keepdims=True)
    o_ref[...] = jnp.sum(s * w_ref[...], axis=1, keepdims=True) * (1.0 / N)


_finalize = pl.pallas_call(
    _finalize_body,
    out_shape=jax.ShapeDtypeStruct((1, 1), jnp.float32),
)


def kernel(word_idxs, table, weights):
    idx = word_idxs.astype(jnp.int32).reshape(NW, B_W)
    partials = _gather_partials(idx, table)
    prod = _finalize(partials, weights.reshape(1, DIM))
    return jnp.reshape(prod, ())
